# baseline (speedup 1.0000x reference)
import functools

import jax
import jax.numpy as jnp
from jax import lax
from jax.experimental import pallas as pl
from jax.experimental.pallas import tpu as pltpu
from jax.experimental.pallas import tpu_sc as plsc

N_NODES = 10000
N_PAD = 10240
CHUNK = 128
NC, NS = 2, 16
NW = NC * NS
BLK = 256
GRID = N_PAD // BLK
SLAB = 16


def _make_agg(cc0: int, cc1: int, width: int):
  assert cc0 % SLAB == 0 and cc1 % SLAB == 0 and SLAB % 2 == 0
  n_parts = 1 if cc1 == 0 else NC
  rows_per_tile = N_PAD // NS
  mesh = plsc.VectorSubcoreMesh(core_axis_name="c", subcore_axis_name="s")
  cparams = pltpu.CompilerParams(use_tc_tiling_on_sc=(width % 128 == 0))

  @functools.partial(
      pl.kernel,
      out_type=jax.ShapeDtypeStruct((n_parts, N_PAD, width), jnp.float32),
      mesh=mesh,
      compiler_params=cparams,
      scratch_types=[
          pltpu.VMEM((SLAB, CHUNK), jnp.int32),
          pltpu.VMEM((SLAB, CHUNK), jnp.int32),
          pltpu.VMEM((2, CHUNK, width), jnp.float32),
          pltpu.VMEM_SHARED((N_PAD, width), jnp.float32),
          pltpu.SemaphoreType.DMA,
          pltpu.SemaphoreType.DMA,
          pltpu.SemaphoreType.DMA,
          pltpu.SemaphoreType.DMA,
          pltpu.SemaphoreType.DMA,
      ],
  )
  def agg(h_hbm, src_hbm, dst_hbm, z_hbm, out_hbm, sidx, didx, rows, acc,
          sem_i, sem_g0, sem_g1, sem_s0, sem_s1):
    c = lax.axis_index("c")
    s = lax.axis_index("s")

    def gather(g, b, sem):
      return pltpu.async_copy(h_hbm.at[sidx.at[g]], rows.at[b], sem)

    def scatter(g, b, sem):
      return pltpu.async_copy(rows.at[b], acc.at[didx.at[g]], sem, add=True)

    active = (c == 0) if cc1 == 0 else (c >= 0)

    @pl.when(active)
    def _():
      r0z = s * rows_per_tile
      if width % 16 == 0:
        @pl.loop(0, CHUNK)
        def _(r):
          @pl.loop(0, width, step=16)
          def _(col):
            rows[0, r, pl.ds(col, 16)] = jnp.zeros((16,), jnp.float32)

        @pl.loop(0, rows_per_tile, step=CHUNK)
        def _(k):
          pltpu.sync_copy(rows.at[0], acc.at[pl.ds(r0z + k, CHUNK)])
      else:
        pltpu.sync_copy(z_hbm.at[pl.ds(r0z, rows_per_tile)],
                        acc.at[pl.ds(r0z, rows_per_tile)])
    plsc.subcore_barrier()

    def run(n_slabs, chunk_base):
      for p in range(n_slabs):
        base = chunk_base + p * SLAB
        pltpu.async_copy(src_hbm.at[pl.ds(base, SLAB)], sidx, sem_i).wait()
        pltpu.async_copy(dst_hbm.at[pl.ds(base, SLAB)], didx, sem_i).wait()
        gather(0, 0, sem_g0)

        @pl.loop(0, SLAB, step=2)
        def _(g):
          pltpu.make_async_copy(h_hbm.at[sidx.at[g]], rows.at[0],
                                sem_g0).wait()
          scatter(g, 0, sem_s0)

          @pl.when(g > 0)
          def _():
            pltpu.make_async_copy(rows.at[1], acc.at[didx.at[g]],
                                  sem_s1).wait()

          gather(g + 1, 1, sem_g1)
          pltpu.make_async_copy(h_hbm.at[sidx.at[g]], rows.at[1],
                                sem_g1).wait()
          scatter(g + 1, 1, sem_s1)

          @pl.when(g + 2 < SLAB)
          def _():
            pltpu.make_async_copy(rows.at[0], acc.at[didx.at[g]],
                                  sem_s0).wait()
            gather(g + 2, 0, sem_g0)

        pltpu.make_async_copy(rows.at[0], acc.at[didx.at[0]], sem_s0).wait()
        pltpu.make_async_copy(rows.at[1], acc.at[didx.at[0]], sem_s1).wait()

    @pl.when(c == 0)
    def _():
      run(cc0 // SLAB, s * cc0)

    if cc1 > 0:
      @pl.when(c == 1)
      def _():
        run(cc1 // SLAB, 16 * cc0 + s * cc1)

    plsc.subcore_barrier()

    @pl.when(active)
    def _():
      @pl.loop(0, rows_per_tile, step=CHUNK)
      def _(k):
        r0 = s * rows_per_tile + k
        pltpu.sync_copy(acc.at[pl.ds(r0, CHUNK)],
                        out_hbm.at[c, pl.ds(r0, CHUNK)])

  return agg


def _make_deg(n_per: int):
  mesh = plsc.VectorSubcoreMesh(core_axis_name="c", subcore_axis_name="s")
  cparams = pltpu.CompilerParams(use_tc_tiling_on_sc=False,
                                 needs_layout_passes=False)

  @functools.partial(
      pl.kernel,
      out_type=jax.ShapeDtypeStruct((NW, N_PAD), jnp.float32),
      mesh=mesh,
      compiler_params=cparams,
      scratch_types=[
          pltpu.VMEM((n_per, CHUNK), jnp.int32),
          pltpu.VMEM((N_PAD,), jnp.float32),
          pltpu.SemaphoreType.DMA,
      ],
  )
  def deg(dst_hbm, out_hbm, didx, hist, sem_i):
    c = lax.axis_index("c")
    s = lax.axis_index("s")
    wid = c * NS + s
    cp_idx = pltpu.async_copy(dst_hbm.at[pl.ds(wid * n_per, n_per)], didx,
                              sem_i)

    @pl.loop(0, N_PAD, step=16)
    def _(r):
      hist[pl.ds(r, 16)] = jnp.zeros((16,), jnp.float32)

    cp_idx.wait()
    ones16 = jnp.ones((16,), jnp.float32)

    @pl.loop(0, n_per)
    def _(g):
      @pl.loop(0, CHUNK, step=16)
      def _(k):
        dvec = didx[g, pl.ds(k, 16)]
        plsc.addupdate_scatter(hist, [dvec], ones16)

    pltpu.sync_copy(hist, out_hbm.at[wid])

  return deg


def _tc1a_body(x_ref, w_ref, h_ref):
  h_ref[...] = jnp.dot(x_ref[...], w_ref[...],
                       preferred_element_type=jnp.float32)


def _tc1b_body(h_ref, degp_ref, hh_ref, dinv_ref):
  d = jnp.sum(degp_ref[...], axis=0) + 1.0
  dinv = lax.rsqrt(d)
  hh_ref[...] = h_ref[...] * dinv[:, None]
  dinv_ref[...] = dinv[:, None]


def _tc2_body(agg_ref, hh_ref, dinv_ref, b1_ref, w2_ref, g_ref):
  a = jnp.sum(agg_ref[...], axis=0)
  dinv = dinv_ref[...]
  l1 = (a + hh_ref[...]) * dinv + b1_ref[...]
  l1 = jnp.maximum(l1, 0.0)
  g = jnp.dot(l1, w2_ref[...], preferred_element_type=jnp.float32)
  g_ref[...] = g * dinv


def _tc3_body(agg_ref, g_ref, dinv_ref, b2_ref, out_ref):
  a = jnp.sum(agg_ref[...], axis=0)
  out_ref[...] = (a + g_ref[...]) * dinv_ref[...] + b2_ref[...]


@jax.jit
def kernel(x, edge_index, W1, b1, W2, b2):
  n, f_in = x.shape
  hidden = W1.shape[1]
  ncls = W2.shape[1]
  e = edge_index.shape[1]

  src = edge_index[0].astype(jnp.int32)
  dst = edge_index[1].astype(jnp.int32)

  t_pairs = -(-e // (NS * 2 * SLAB * CHUNK)) * 2 * SLAB
  total_chunks = NS * t_pairs
  e_pad = total_chunks * CHUNK
  src3 = jnp.concatenate(
      [src, jnp.zeros((e_pad - e,), jnp.int32)]).reshape(total_chunks, CHUNK)
  dst3 = jnp.concatenate(
      [dst, jnp.full((e_pad - e,), N_PAD - 1, jnp.int32)]).reshape(
          total_chunks, CHUNK)

  w2_w = 8
  x_pad = jnp.zeros((N_PAD, f_in), jnp.float32).at[:n].set(x)
  w2p = jnp.zeros((hidden, w2_w), jnp.float32).at[:, :ncls].set(W2)
  b1_2d = b1[None, :]
  b2p = jnp.zeros((1, w2_w), jnp.float32).at[0, :ncls].set(b2)
  z_dummy = jnp.zeros((8,), jnp.float32)
  z_narrow = jnp.zeros((N_PAD, w2_w), jnp.float32)

  degp = _make_deg(total_chunks // NW)(dst3)

  h = pl.pallas_call(
      _tc1a_body,
      grid=(GRID,),
      in_specs=[
          pl.BlockSpec((BLK, f_in), lambda i: (i, 0)),
          pl.BlockSpec((f_in, hidden), lambda i: (0, 0)),
      ],
      out_specs=pl.BlockSpec((BLK, hidden), lambda i: (i, 0)),
      out_shape=jax.ShapeDtypeStruct((N_PAD, hidden), jnp.float32),
  )(x_pad, W1)

  hh, dinv = pl.pallas_call(
      _tc1b_body,
      grid=(GRID,),
      in_specs=[
          pl.BlockSpec((BLK, hidden), lambda i: (i, 0)),
          pl.BlockSpec((NW, BLK), lambda i: (0, i)),
      ],
      out_specs=[
          pl.BlockSpec((BLK, hidden), lambda i: (i, 0)),
          pl.BlockSpec((BLK, 1), lambda i: (i, 0)),
      ],
      out_shape=[
          jax.ShapeDtypeStruct((N_PAD, hidden), jnp.float32),
          jax.ShapeDtypeStruct((N_PAD, 1), jnp.float32),
      ],
  )(h, degp)

  agg1 = _make_agg(9 * t_pairs // 10, t_pairs // 10, f_in)(
      hh, src3, dst3, z_dummy)

  g = pl.pallas_call(
      _tc2_body,
      grid=(GRID,),
      in_specs=[
          pl.BlockSpec((NC, BLK, hidden), lambda i: (0, i, 0)),
          pl.BlockSpec((BLK, hidden), lambda i: (i, 0)),
          pl.BlockSpec((BLK, 1), lambda i: (i, 0)),
          pl.BlockSpec((1, hidden), lambda i: (0, 0)),
          pl.BlockSpec((hidden, w2_w), lambda i: (0, 0)),
      ],
      out_specs=pl.BlockSpec((BLK, w2_w), lambda i: (i, 0)),
      out_shape=jax.ShapeDtypeStruct((N_PAD, w2_w), jnp.float32),
  )(agg1, hh, dinv, b1_2d, w2p)

  agg2 = _make_agg(t_pairs // 2, t_pairs // 2, w2_w)(g, src3, dst3, z_narrow)

  out = pl.pallas_call(
      _tc3_body,
      grid=(GRID,),
      in_specs=[
          pl.BlockSpec((NC, BLK, w2_w), lambda i: (0, i, 0)),
          pl.BlockSpec((BLK, w2_w), lambda i: (i, 0)),
          pl.BlockSpec((BLK, 1), lambda i: (i, 0)),
          pl.BlockSpec((1, w2_w), lambda i: (0, 0)),
      ],
      out_specs=pl.BlockSpec((BLK, w2_w), lambda i: (i, 0)),
      out_shape=jax.ShapeDtypeStruct((N_PAD, w2_w), jnp.float32),
  )(agg2, g, dinv, b2p)

  return out[:n, :ncls]


# speedup vs baseline: 1.0044x; 1.0044x over previous
import functools

import jax
import jax.numpy as jnp
from jax import lax
from jax.experimental import pallas as pl
from jax.experimental.pallas import tpu as pltpu
from jax.experimental.pallas import tpu_sc as plsc

N_NODES = 10000
N_PAD = 10240
CHUNK = 128
NC, NS = 2, 16
NW = NC * NS
BLK = 256
GRID = N_PAD // BLK
SLAB = 40


def _make_agg(cc0: int, cc1: int, width: int):
  assert cc0 % SLAB == 0 and cc1 % SLAB == 0 and SLAB % 2 == 0
  n_parts = 1 if cc1 == 0 else NC
  rows_per_tile = N_PAD // NS
  mesh = plsc.VectorSubcoreMesh(core_axis_name="c", subcore_axis_name="s")
  cparams = pltpu.CompilerParams(use_tc_tiling_on_sc=(width % 128 == 0))

  @functools.partial(
      pl.kernel,
      out_type=jax.ShapeDtypeStruct((n_parts, N_PAD, width), jnp.float32),
      mesh=mesh,
      compiler_params=cparams,
      scratch_types=[
          pltpu.VMEM((SLAB, CHUNK), jnp.int32),
          pltpu.VMEM((SLAB, CHUNK), jnp.int32),
          pltpu.VMEM((2, CHUNK, width), jnp.float32),
          pltpu.VMEM_SHARED((N_PAD, width), jnp.float32),
          pltpu.SemaphoreType.DMA,
          pltpu.SemaphoreType.DMA,
          pltpu.SemaphoreType.DMA,
          pltpu.SemaphoreType.DMA,
          pltpu.SemaphoreType.DMA,
      ],
  )
  def agg(h_hbm, src_hbm, dst_hbm, z_hbm, out_hbm, sidx, didx, rows, acc,
          sem_i, sem_g0, sem_g1, sem_s0, sem_s1):
    c = lax.axis_index("c")
    s = lax.axis_index("s")

    def gather(g, b, sem):
      return pltpu.async_copy(h_hbm.at[sidx.at[g]], rows.at[b], sem)

    def scatter(g, b, sem):
      return pltpu.async_copy(rows.at[b], acc.at[didx.at[g]], sem, add=True)

    active = (c == 0) if cc1 == 0 else (c >= 0)

    @pl.when(active)
    def _():
      r0z = s * rows_per_tile
      if width % 16 == 0:
        @pl.loop(0, CHUNK)
        def _(r):
          @pl.loop(0, width, step=16)
          def _(col):
            rows[0, r, pl.ds(col, 16)] = jnp.zeros((16,), jnp.float32)

        @pl.loop(0, rows_per_tile, step=CHUNK)
        def _(k):
          pltpu.sync_copy(rows.at[0], acc.at[pl.ds(r0z + k, CHUNK)])
      else:
        pltpu.sync_copy(z_hbm.at[pl.ds(r0z, rows_per_tile)],
                        acc.at[pl.ds(r0z, rows_per_tile)])
    plsc.subcore_barrier()

    def run(n_slabs, chunk_base):
      for p in range(n_slabs):
        base = chunk_base + p * SLAB
        pltpu.async_copy(src_hbm.at[pl.ds(base, SLAB)], sidx, sem_i).wait()
        pltpu.async_copy(dst_hbm.at[pl.ds(base, SLAB)], didx, sem_i).wait()
        gather(0, 0, sem_g0)

        @pl.loop(0, SLAB, step=2)
        def _(g):
          pltpu.make_async_copy(h_hbm.at[sidx.at[g]], rows.at[0],
                                sem_g0).wait()
          scatter(g, 0, sem_s0)

          @pl.when(g > 0)
          def _():
            pltpu.make_async_copy(rows.at[1], acc.at[didx.at[g]],
                                  sem_s1).wait()

          gather(g + 1, 1, sem_g1)
          pltpu.make_async_copy(h_hbm.at[sidx.at[g]], rows.at[1],
                                sem_g1).wait()
          scatter(g + 1, 1, sem_s1)

          @pl.when(g + 2 < SLAB)
          def _():
            pltpu.make_async_copy(rows.at[0], acc.at[didx.at[g]],
                                  sem_s0).wait()
            gather(g + 2, 0, sem_g0)

        pltpu.make_async_copy(rows.at[0], acc.at[didx.at[0]], sem_s0).wait()
        pltpu.make_async_copy(rows.at[1], acc.at[didx.at[0]], sem_s1).wait()

    @pl.when(c == 0)
    def _():
      run(cc0 // SLAB, s * cc0)

    if cc1 > 0:
      @pl.when(c == 1)
      def _():
        run(cc1 // SLAB, 16 * cc0 + s * cc1)

    plsc.subcore_barrier()

    @pl.when(active)
    def _():
      @pl.loop(0, rows_per_tile, step=CHUNK)
      def _(k):
        r0 = s * rows_per_tile + k
        pltpu.sync_copy(acc.at[pl.ds(r0, CHUNK)],
                        out_hbm.at[c, pl.ds(r0, CHUNK)])

  return agg


def _make_deg(n_per: int):
  mesh = plsc.VectorSubcoreMesh(core_axis_name="c", subcore_axis_name="s")
  cparams = pltpu.CompilerParams(use_tc_tiling_on_sc=False,
                                 needs_layout_passes=False)

  @functools.partial(
      pl.kernel,
      out_type=jax.ShapeDtypeStruct((NW, N_PAD), jnp.float32),
      mesh=mesh,
      compiler_params=cparams,
      scratch_types=[
          pltpu.VMEM((n_per, CHUNK), jnp.int32),
          pltpu.VMEM((N_PAD,), jnp.float32),
          pltpu.SemaphoreType.DMA,
      ],
  )
  def deg(dst_hbm, out_hbm, didx, hist, sem_i):
    c = lax.axis_index("c")
    s = lax.axis_index("s")
    wid = c * NS + s
    cp_idx = pltpu.async_copy(dst_hbm.at[pl.ds(wid * n_per, n_per)], didx,
                              sem_i)

    @pl.loop(0, N_PAD, step=16)
    def _(r):
      hist[pl.ds(r, 16)] = jnp.zeros((16,), jnp.float32)

    cp_idx.wait()
    ones16 = jnp.ones((16,), jnp.float32)

    @pl.loop(0, n_per)
    def _(g):
      @pl.loop(0, CHUNK, step=16)
      def _(k):
        dvec = didx[g, pl.ds(k, 16)]
        plsc.addupdate_scatter(hist, [dvec], ones16)

    pltpu.sync_copy(hist, out_hbm.at[wid])

  return deg


def _tc1a_body(x_ref, w_ref, h_ref):
  h_ref[...] = jnp.dot(x_ref[...], w_ref[...],
                       preferred_element_type=jnp.float32)


def _tc1b_body(h_ref, degp_ref, hh_ref, dinv_ref):
  d = jnp.sum(degp_ref[...], axis=0) + 1.0
  dinv = lax.rsqrt(d)
  hh_ref[...] = h_ref[...] * dinv[:, None]
  dinv_ref[...] = dinv[:, None]


def _tc2_body(agg_ref, hh_ref, dinv_ref, b1_ref, w2_ref, g_ref):
  a = jnp.sum(agg_ref[...], axis=0)
  dinv = dinv_ref[...]
  l1 = (a + hh_ref[...]) * dinv + b1_ref[...]
  l1 = jnp.maximum(l1, 0.0)
  g = jnp.dot(l1, w2_ref[...], preferred_element_type=jnp.float32)
  g_ref[...] = g * dinv


def _tc3_body(agg_ref, g_ref, dinv_ref, b2_ref, out_ref):
  a = jnp.sum(agg_ref[...], axis=0)
  out_ref[...] = (a + g_ref[...]) * dinv_ref[...] + b2_ref[...]


@jax.jit
def kernel(x, edge_index, W1, b1, W2, b2):
  n, f_in = x.shape
  hidden = W1.shape[1]
  ncls = W2.shape[1]
  e = edge_index.shape[1]

  src = edge_index[0].astype(jnp.int32)
  dst = edge_index[1].astype(jnp.int32)

  t_pairs = -(-e // (NS * 2 * SLAB * CHUNK)) * 2 * SLAB
  total_chunks = NS * t_pairs
  e_pad = total_chunks * CHUNK
  src3 = jnp.concatenate(
      [src, jnp.zeros((e_pad - e,), jnp.int32)]).reshape(total_chunks, CHUNK)
  dst3 = jnp.concatenate(
      [dst, jnp.full((e_pad - e,), N_PAD - 1, jnp.int32)]).reshape(
          total_chunks, CHUNK)

  w2_w = 8
  x_pad = jnp.zeros((N_PAD, f_in), jnp.float32).at[:n].set(x)
  w2p = jnp.zeros((hidden, w2_w), jnp.float32).at[:, :ncls].set(W2)
  b1_2d = b1[None, :]
  b2p = jnp.zeros((1, w2_w), jnp.float32).at[0, :ncls].set(b2)
  z_dummy = jnp.zeros((8,), jnp.float32)
  z_narrow = jnp.zeros((N_PAD, w2_w), jnp.float32)

  degp = _make_deg(total_chunks // NW)(dst3)

  h = pl.pallas_call(
      _tc1a_body,
      grid=(GRID,),
      in_specs=[
          pl.BlockSpec((BLK, f_in), lambda i: (i, 0)),
          pl.BlockSpec((f_in, hidden), lambda i: (0, 0)),
      ],
      out_specs=pl.BlockSpec((BLK, hidden), lambda i: (i, 0)),
      out_shape=jax.ShapeDtypeStruct((N_PAD, hidden), jnp.float32),
  )(x_pad, W1)

  hh, dinv = pl.pallas_call(
      _tc1b_body,
      grid=(GRID,),
      in_specs=[
          pl.BlockSpec((BLK, hidden), lambda i: (i, 0)),
          pl.BlockSpec((NW, BLK), lambda i: (0, i)),
      ],
      out_specs=[
          pl.BlockSpec((BLK, hidden), lambda i: (i, 0)),
          pl.BlockSpec((BLK, 1), lambda i: (i, 0)),
      ],
      out_shape=[
          jax.ShapeDtypeStruct((N_PAD, hidden), jnp.float32),
          jax.ShapeDtypeStruct((N_PAD, 1), jnp.float32),
      ],
  )(h, degp)

  agg1 = _make_agg(3 * t_pairs // 4, t_pairs // 4, f_in)(
      hh, src3, dst3, z_dummy)

  g = pl.pallas_call(
      _tc2_body,
      grid=(GRID,),
      in_specs=[
          pl.BlockSpec((NC, BLK, hidden), lambda i: (0, i, 0)),
          pl.BlockSpec((BLK, hidden), lambda i: (i, 0)),
          pl.BlockSpec((BLK, 1), lambda i: (i, 0)),
          pl.BlockSpec((1, hidden), lambda i: (0, 0)),
          pl.BlockSpec((hidden, w2_w), lambda i: (0, 0)),
      ],
      out_specs=pl.BlockSpec((BLK, w2_w), lambda i: (i, 0)),
      out_shape=jax.ShapeDtypeStruct((N_PAD, w2_w), jnp.float32),
  )(agg1, hh, dinv, b1_2d, w2p)

  agg2 = _make_agg(t_pairs // 2, t_pairs // 2, w2_w)(g, src3, dst3, z_narrow)

  out = pl.pallas_call(
      _tc3_body,
      grid=(GRID,),
      in_specs=[
          pl.BlockSpec((NC, BLK, w2_w), lambda i: (0, i, 0)),
          pl.BlockSpec((BLK, w2_w), lambda i: (i, 0)),
          pl.BlockSpec((BLK, 1), lambda i: (i, 0)),
          pl.BlockSpec((1, w2_w), lambda i: (0, 0)),
      ],
      out_specs=pl.BlockSpec((BLK, w2_w), lambda i: (i, 0)),
      out_shape=jax.ShapeDtypeStruct((N_PAD, w2_w), jnp.float32),
  )(agg2, g, dinv, b2p)

  return out[:n, :ncls]
